# Initial kernel scaffold; baseline (speedup 1.0000x reference)
import jax
import jax.numpy as jnp
from jax.experimental import pallas as pl


def kernel(feat, w1, w2, b2, ws, bs, bn_gamma, bn_beta, ln_gamma, ln_beta):
    raise NotImplementedError("write your pallas kernel here")


# speedup vs baseline: 1.2199x; 1.2199x over previous
import functools

import jax
import jax.numpy as jnp
from jax import lax
from jax.experimental import pallas as pl
from jax.experimental.pallas import tpu as pltpu

EPS = 1e-5


def _round_up(x, m):
    return (x + m - 1) // m * m


def _gram_kernel(feat_ref, c_ref, rs_ref):
    i = pl.program_id(1)

    @pl.when(i == 0)
    def _():
        c_ref[...] = jnp.zeros_like(c_ref)
        rs_ref[...] = jnp.zeros_like(rs_ref)

    fb = feat_ref[...].astype(jnp.bfloat16)
    c_ref[...] += lax.dot_general(
        fb, fb, (((0,), (0,)), ((), ())), preferred_element_type=jnp.float32)
    tm, f_sz = feat_ref.shape
    rs_ref[...] += jnp.sum(
        fb.astype(jnp.float32).reshape(tm // 8, 8, f_sz), axis=0)


def _stats_kernel(c_ref, rs_ref, w1t_ref, st_ref, *, n_rows):
    inv_n = 1.0 / n_rows
    c = c_ref[0] + c_ref[1]
    rs = rs_ref[0] + rs_ref[1]
    w1t = w1t_ref[...].astype(jnp.float32)

    d = jnp.dot(c, w1t, preferred_element_type=jnp.float32)
    e2 = jnp.sum(w1t * d, axis=0, keepdims=True) * inv_n

    m8 = jnp.dot(rs, w1t, preferred_element_type=jnp.float32)
    mean = jnp.sum(m8, axis=0, keepdims=True) * inv_n

    var = e2 - mean * mean
    inv_std = lax.rsqrt(jnp.maximum(var, 0.0) + EPS)

    st_ref[...] = jnp.zeros_like(st_ref)
    st_ref[0:1, :] = mean
    st_ref[1:2, :] = inv_std


def _apply_kernel(feat_ref, w_ref, w2_ref, p_ref, st_ref, out_ref):
    units = w2_ref.shape[0]
    fb = feat_ref[...].astype(jnp.bfloat16)
    hs = jnp.dot(fb, w_ref[...], preferred_element_type=jnp.float32)
    h = hs[:, :units]
    s = hs[:, units:]

    bias = p_ref[0:1, :]
    bn_g = p_ref[1:2, :]
    bn_b = p_ref[2:3, :]
    ln_g = p_ref[3:4, :]
    ln_b = p_ref[4:5, :]
    mean = st_ref[0:1, :]
    inv_std = st_ref[1:2, :]

    h = (h - mean) * (inv_std * bn_g) + bn_b
    hb = jnp.maximum(h, 0.0).astype(jnp.bfloat16)

    f = jnp.dot(hb, w2_ref[...], preferred_element_type=jnp.float32) + s + bias

    mu = jnp.mean(f, axis=-1, keepdims=True)
    d = f - mu
    v = jnp.mean(d * d, axis=-1, keepdims=True)
    out_ref[...] = (d * lax.rsqrt(v + EPS) * ln_g + ln_b).astype(out_ref.dtype)


def kernel(feat, w1, w2, b2, ws, bs, bn_gamma, bn_beta, ln_gamma, ln_beta):
    n, f_sz = feat.shape
    u = w2.shape[0]

    w_feat = jnp.concatenate([w1.T, ws.T], axis=1).astype(jnp.bfloat16)
    w2t = w2.T.astype(jnp.bfloat16)
    pvec = jnp.zeros((8, u), jnp.float32)
    pvec = pvec.at[0].set(b2 + bs)
    pvec = pvec.at[1].set(bn_gamma)
    pvec = pvec.at[2].set(bn_beta)
    pvec = pvec.at[3].set(ln_gamma)
    pvec = pvec.at[4].set(ln_beta)

    tm1 = 2048
    tm2 = 1024
    vmem_limit = 48 * 1024 * 1024

    n_pad = _round_up(n, 2 * tm1)
    feat_p = jnp.pad(feat, ((0, n_pad - n), (0, 0))) if n_pad != n else feat
    half = n_pad // (2 * tm1)

    c_part, rs_part = pl.pallas_call(
        _gram_kernel,
        out_shape=(
            jax.ShapeDtypeStruct((2, f_sz, f_sz), jnp.float32),
            jax.ShapeDtypeStruct((2, 8, f_sz), jnp.float32),
        ),
        grid=(2, half),
        in_specs=[
            pl.BlockSpec((tm1, f_sz), lambda c, i: (c * half + i, 0)),
        ],
        out_specs=(
            pl.BlockSpec((None, f_sz, f_sz), lambda c, i: (c, 0, 0)),
            pl.BlockSpec((None, 8, f_sz), lambda c, i: (c, 0, 0)),
        ),
        compiler_params=pltpu.CompilerParams(
            dimension_semantics=("parallel", "arbitrary"),
            vmem_limit_bytes=vmem_limit),
    )(feat_p)

    stats = pl.pallas_call(
        functools.partial(_stats_kernel, n_rows=float(n)),
        out_shape=jax.ShapeDtypeStruct((8, u), jnp.float32),
        grid=(1,),
        in_specs=[
            pl.BlockSpec((2, f_sz, f_sz), lambda i: (0, 0, 0)),
            pl.BlockSpec((2, 8, f_sz), lambda i: (0, 0, 0)),
            pl.BlockSpec((f_sz, u), lambda i: (0, 0)),
        ],
        out_specs=pl.BlockSpec((8, u), lambda i: (0, 0)),
        compiler_params=pltpu.CompilerParams(
            dimension_semantics=("arbitrary",),
            vmem_limit_bytes=vmem_limit),
    )(c_part, rs_part, w_feat)

    n2 = _round_up(n, tm2)
    n_tiles = n2 // tm2
    feat2 = feat_p[:n2]
    out = pl.pallas_call(
        _apply_kernel,
        out_shape=jax.ShapeDtypeStruct((n2, u), feat.dtype),
        grid=(n_tiles,),
        in_specs=[
            pl.BlockSpec((tm2, f_sz), lambda i: (i, 0)),
            pl.BlockSpec((f_sz, 2 * u), lambda i: (0, 0)),
            pl.BlockSpec((u, u), lambda i: (0, 0)),
            pl.BlockSpec((8, u), lambda i: (0, 0)),
            pl.BlockSpec((8, u), lambda i: (0, 0)),
        ],
        out_specs=pl.BlockSpec((tm2, u), lambda i: (i, 0)),
        compiler_params=pltpu.CompilerParams(
            dimension_semantics=("parallel",),
            vmem_limit_bytes=vmem_limit),
    )(feat2, w_feat, w2t, pvec, stats)

    return out[:n] if n2 != n else out
